# initial kernel scaffold (unmeasured)
import jax
import jax.numpy as jnp
from jax import lax
from jax.experimental import pallas as pl
from jax.experimental.pallas import tpu as pltpu

_BLOCK_M = 256
_EPS = 1e-5


def kernel(x, dy, gamma):
    m, d = x.shape
    n_blocks = m // _BLOCK_M

    def body(x_ref, dy_ref, gamma_ref, out_ref, acc_ref, recv_ref,
             send_sem, recv_sem):
        i = pl.program_id(0)

        xb = x_ref[:, :]
        dyb = dy_ref[:, :]
        mu = jnp.mean(xb, axis=1, keepdims=True)
        var = jnp.mean((xb - mu) ** 2, axis=1, keepdims=True)
        xhat = (xb - mu) * lax.rsqrt(var + _EPS)
        dgamma = jnp.sum(dyb * xhat, axis=0, keepdims=True)
        dbeta = jnp.sum(dyb, axis=0, keepdims=True)

        @pl.when(i == 0)
        def _():
            acc_ref[0:1, :] = dgamma
            acc_ref[1:2, :] = dbeta

        @pl.when(i > 0)
        def _():
            acc_ref[0:1, :] = acc_ref[0:1, :] + dgamma
            acc_ref[1:2, :] = acc_ref[1:2, :] + dbeta

        @pl.when(i == n_blocks - 1)
        def _():
            my_x = lax.axis_index("x")
            my_y = lax.axis_index("y")
            my_z = lax.axis_index("z")
            peer = (my_x, 1 - my_y, my_z)

            barrier_sem = pltpu.get_barrier_semaphore()
            pl.semaphore_signal(
                barrier_sem, inc=1, device_id=peer,
                device_id_type=pl.DeviceIdType.MESH,
            )
            pl.semaphore_wait(barrier_sem, 1)

            rdma = pltpu.make_async_remote_copy(
                src_ref=acc_ref,
                dst_ref=recv_ref,
                send_sem=send_sem,
                recv_sem=recv_sem,
                device_id=peer,
                device_id_type=pl.DeviceIdType.MESH,
            )
            rdma.start()
            rdma.wait()
            out_ref[:, :] = acc_ref[:, :] + recv_ref[:, :]

    return pl.pallas_call(
        body,
        grid=(n_blocks,),
        out_shape=jax.ShapeDtypeStruct((2, d), jnp.float32),
        in_specs=[
            pl.BlockSpec((_BLOCK_M, d), lambda i: (i, 0)),
            pl.BlockSpec((_BLOCK_M, d), lambda i: (i, 0)),
            pl.BlockSpec(memory_space=pltpu.ANY),
        ],
        out_specs=pl.BlockSpec((2, d), lambda i: (0, 0)),
        scratch_shapes=[
            pltpu.VMEM((2, d), jnp.float32),
            pltpu.VMEM((2, d), jnp.float32),
            pltpu.SemaphoreType.DMA,
            pltpu.SemaphoreType.DMA,
        ],
        compiler_params=pltpu.CompilerParams(collective_id=0),
    )(x, dy, gamma)


# baseline (device time: 21599 ns/iter reference)
import jax
import jax.numpy as jnp
from jax import lax
from jax.experimental import pallas as pl
from jax.experimental.pallas import tpu as pltpu

_BLOCK_M = 256
_EPS = 1e-5


def kernel(x, dy, gamma):
    m, d = x.shape
    n_blocks = m // _BLOCK_M

    def body(x_ref, dy_ref, gamma_ref, out_ref, acc_ref, recv_ref,
             send_sem, recv_sem):
        i = pl.program_id(0)

        xb = x_ref[:, :]
        dyb = dy_ref[:, :]
        mu = jnp.mean(xb, axis=1, keepdims=True)
        var = jnp.mean((xb - mu) ** 2, axis=1, keepdims=True)
        xhat = (xb - mu) * lax.rsqrt(var + _EPS)
        dgamma = jnp.sum(dyb * xhat, axis=0, keepdims=True)
        dbeta = jnp.sum(dyb, axis=0, keepdims=True)

        @pl.when(i == 0)
        def _():
            acc_ref[0:1, :] = dgamma
            acc_ref[1:2, :] = dbeta

        @pl.when(i > 0)
        def _():
            acc_ref[0:1, :] = acc_ref[0:1, :] + dgamma
            acc_ref[1:2, :] = acc_ref[1:2, :] + dbeta

        @pl.when(i == n_blocks - 1)
        def _():
            my_x = lax.axis_index("x")
            my_y = lax.axis_index("y")
            my_z = lax.axis_index("z")
            peer = (my_x, 1 - my_y, my_z)

            barrier_sem = pltpu.get_barrier_semaphore()
            pl.semaphore_signal(
                barrier_sem, inc=1, device_id=peer,
                device_id_type=pl.DeviceIdType.MESH,
            )
            pl.semaphore_wait(barrier_sem, 1)

            rdma = pltpu.make_async_remote_copy(
                src_ref=acc_ref,
                dst_ref=recv_ref,
                send_sem=send_sem,
                recv_sem=recv_sem,
                device_id=peer,
                device_id_type=pl.DeviceIdType.MESH,
            )
            rdma.start()
            rdma.wait()
            out_ref[:, :] = acc_ref[:, :] + recv_ref[:, :]

    return pl.pallas_call(
        body,
        grid=(n_blocks,),
        out_shape=jax.ShapeDtypeStruct((2, d), jnp.float32),
        in_specs=[
            pl.BlockSpec((_BLOCK_M, d), lambda i: (i, 0)),
            pl.BlockSpec((_BLOCK_M, d), lambda i: (i, 0)),
            pl.BlockSpec(memory_space=pl.ANY),
        ],
        out_specs=pl.BlockSpec((2, d), lambda i: (0, 0)),
        scratch_shapes=[
            pltpu.VMEM((2, d), jnp.float32),
            pltpu.VMEM((2, d), jnp.float32),
            pltpu.SemaphoreType.DMA,
            pltpu.SemaphoreType.DMA,
        ],
        compiler_params=pltpu.CompilerParams(collective_id=0),
    )(x, dy, gamma)
